# baseline (speedup 1.0000x reference)
import functools
import math

import jax
import jax.numpy as jnp
from jax import lax
from jax.experimental import pallas as pl
from jax.experimental.pallas import tpu as pltpu
from jax.experimental.pallas import tpu_sc as plsc

VOCAB = 1000000
D_MODEL = 64
BATCH = 4096
HIST = 200

NC = 2
NS = 16
NW = NC * NS
B = BATCH * HIST
C = 128
G4 = 4
R = C * G4
RPW = B // NW
NCH = RPW // C
NG = RPW // R
NBUF = 3
SCALE = math.sqrt(float(D_MODEL))

VBLK = 512
NBLK = (VOCAB + VBLK - 1) // VBLK
PK_ROWS = NBLK * (VBLK // 2)

_mesh = plsc.VectorSubcoreMesh(core_axis_name="c", subcore_axis_name="s")


def _untile_body(in_ref, out_ref):
    blk = in_ref[...]
    m = jnp.concatenate([blk[:, :256], blk[:, 256:]], axis=0)
    out_ref[...] = m.T * SCALE


_untile = pl.pallas_call(
    _untile_body,
    grid=(NBLK,),
    in_specs=[pl.BlockSpec((D_MODEL, VBLK), lambda i: (0, i))],
    out_specs=pl.BlockSpec((VBLK // 2, 2 * D_MODEL), lambda i: (i, 0)),
    out_shape=jax.ShapeDtypeStruct((PK_ROWS, 2 * D_MODEL), jnp.float32),
)


@functools.partial(
    pl.kernel,
    out_type=jax.ShapeDtypeStruct((B, 2 * D_MODEL), jnp.float32),
    mesh=_mesh,
    compiler_params=pltpu.CompilerParams(use_tc_tiling_on_sc=False),
    scratch_types=[
        pltpu.VMEM((NCH, C), jnp.int32),
        pltpu.VMEM((NBUF, R, D_MODEL), jnp.float32),
        pltpu.SemaphoreType.DMA,
        pltpu.SemaphoreType.DMA,
        pltpu.SemaphoreType.DMA,
        pltpu.SemaphoreType.DMA,
        pltpu.SemaphoreType.DMA,
        pltpu.SemaphoreType.DMA,
    ],
)
def _emb_kernel(lut_hbm, idx_hbm, out_hbm, idx_v, rows_v, g0, g1, g2, s0, s1, s2):
    gsem = (g0, g1, g2)
    ssem = (s0, s1, s2)
    wid = lax.axis_index("s") * NC + lax.axis_index("c")
    row0 = wid * RPW

    pltpu.sync_copy(idx_hbm.at[pl.ds(wid * NCH, NCH)], idx_v)

    def fire_group(g, b):
        for j in range(G4):
            pltpu.async_copy(
                lut_hbm.at[idx_v.at[g * G4 + j]],
                rows_v.at[b, pl.ds(j * C, C)],
                gsem[b],
            )

    def wait_group(g, b):
        for j in range(G4):
            pltpu.make_async_copy(
                lut_hbm.at[idx_v.at[g * G4 + j]],
                rows_v.at[b, pl.ds(j * C, C)],
                gsem[b],
            ).wait()

    def fire_scatter(g, b):
        pltpu.async_copy(
            rows_v.at[b],
            out_hbm.at[pl.ds(row0 + g * R, R), pl.ds(0, D_MODEL)],
            ssem[b],
        )

    def wait_scatter(g, b):
        pltpu.make_async_copy(
            rows_v.at[b],
            out_hbm.at[pl.ds(row0 + g * R, R), pl.ds(0, D_MODEL)],
            ssem[b],
        ).wait()

    fire_group(0, 0)
    fire_group(1, 1)

    def step(t, _):
        for b in range(NBUF):
            g = t * NBUF + b

            @pl.when(g < NG)
            def _():
                wait_group(g, b)
                fire_scatter(g, b)
                bn = (b + 2) % NBUF

                @pl.when(jnp.logical_and(g >= 1, g + 2 < NG))
                def _():
                    wait_scatter(g - 1, bn)

                @pl.when(g + 2 < NG)
                def _():
                    fire_group(g + 2, bn)
        return 0

    lax.fori_loop(0, (NG + NBUF - 1) // NBUF, step, 0)

    for g in (NG - 3, NG - 2, NG - 1):
        wait_scatter(g, g % NBUF)


def kernel(x, lut):
    lut_pk = _untile(jnp.transpose(lut))
    lut2 = lut_pk.reshape(2 * PK_ROWS, D_MODEL)
    xi = x.astype(jnp.int32)
    r = ((xi >> 9) << 9) | ((xi & 255) << 1) | ((xi >> 8) & 1)
    idx2d = r.reshape(B // C, C)
    out = _emb_kernel(lut2, idx2d)
    return out[:, :D_MODEL].reshape(BATCH, HIST, D_MODEL)


# speedup vs baseline: 2.6940x; 2.6940x over previous
import functools
import math

import jax
import jax.numpy as jnp
from jax import lax
from jax.experimental import pallas as pl
from jax.experimental.pallas import tpu as pltpu
from jax.experimental.pallas import tpu_sc as plsc

VOCAB = 1000000
D_MODEL = 64
BATCH = 4096
HIST = 200

NC = 2
NS = 16
NW = NC * NS
B = BATCH * HIST
C = 128
G4 = 4
R = C * G4
RPW = B // NW
NCH = RPW // C
NG = RPW // R
NBUF = 3
SCALE = math.sqrt(float(D_MODEL))

VBLK = 8192
NBLK = (VOCAB + VBLK - 1) // VBLK
PK_ROWS = NBLK * (VBLK // 2)

_mesh = plsc.VectorSubcoreMesh(core_axis_name="c", subcore_axis_name="s")


def _untile_body(in_ref, out_ref):
    blk = in_ref[...]
    h = VBLK // 2
    m = jnp.concatenate([blk[:, :h], blk[:, h:]], axis=0)
    out_ref[...] = m.T * SCALE


_untile = pl.pallas_call(
    _untile_body,
    grid=(NBLK,),
    in_specs=[pl.BlockSpec((D_MODEL, VBLK), lambda i: (0, i))],
    out_specs=pl.BlockSpec((VBLK // 2, 2 * D_MODEL), lambda i: (i, 0)),
    out_shape=jax.ShapeDtypeStruct((PK_ROWS, 2 * D_MODEL), jnp.float32),
)


@functools.partial(
    pl.kernel,
    out_type=jax.ShapeDtypeStruct((B, 2 * D_MODEL), jnp.float32),
    mesh=_mesh,
    compiler_params=pltpu.CompilerParams(use_tc_tiling_on_sc=False),
    scratch_types=[
        pltpu.VMEM((NCH, C), jnp.int32),
        pltpu.VMEM((NBUF, R, D_MODEL), jnp.float32),
        pltpu.SemaphoreType.DMA,
        pltpu.SemaphoreType.DMA,
        pltpu.SemaphoreType.DMA,
        pltpu.SemaphoreType.DMA,
        pltpu.SemaphoreType.DMA,
        pltpu.SemaphoreType.DMA,
    ],
)
def _emb_kernel(lut_hbm, idx_hbm, out_hbm, idx_v, rows_v, g0, g1, g2, s0, s1, s2):
    gsem = (g0, g1, g2)
    ssem = (s0, s1, s2)
    wid = lax.axis_index("s") * NC + lax.axis_index("c")
    row0 = wid * RPW

    pltpu.sync_copy(idx_hbm.at[pl.ds(wid * NCH, NCH)], idx_v)

    def fire_group(g, b):
        for j in range(G4):
            pltpu.async_copy(
                lut_hbm.at[idx_v.at[g * G4 + j]],
                rows_v.at[b, pl.ds(j * C, C)],
                gsem[b],
            )

    def wait_group(g, b):
        for j in range(G4):
            pltpu.make_async_copy(
                lut_hbm.at[idx_v.at[g * G4 + j]],
                rows_v.at[b, pl.ds(j * C, C)],
                gsem[b],
            ).wait()

    def fire_scatter(g, b):
        pltpu.async_copy(
            rows_v.at[b],
            out_hbm.at[pl.ds(row0 + g * R, R), pl.ds(0, D_MODEL)],
            ssem[b],
        )

    def wait_scatter(g, b):
        pltpu.make_async_copy(
            rows_v.at[b],
            out_hbm.at[pl.ds(row0 + g * R, R), pl.ds(0, D_MODEL)],
            ssem[b],
        ).wait()

    fire_group(0, 0)
    fire_group(1, 1)

    def step(t, _):
        for b in range(NBUF):
            g = t * NBUF + b

            @pl.when(g < NG)
            def _():
                wait_group(g, b)
                fire_scatter(g, b)
                bn = (b + 2) % NBUF

                @pl.when(jnp.logical_and(g >= 1, g + 2 < NG))
                def _():
                    wait_scatter(g - 1, bn)

                @pl.when(g + 2 < NG)
                def _():
                    fire_group(g + 2, bn)
        return 0

    lax.fori_loop(0, (NG + NBUF - 1) // NBUF, step, 0)

    for g in (NG - 3, NG - 2, NG - 1):
        wait_scatter(g, g % NBUF)


def kernel(x, lut):
    lut_pk = _untile(jnp.transpose(lut))
    lut2 = lut_pk.reshape(2 * PK_ROWS, D_MODEL)
    xi = x.astype(jnp.int32)
    r = ((xi >> 13) << 13) | ((xi & 4095) << 1) | ((xi >> 12) & 1)
    idx2d = r.reshape(B // C, C)
    out = _emb_kernel(lut2, idx2d)
    return out[:, :D_MODEL].reshape(BATCH, HIST, D_MODEL)


# speedup vs baseline: 2.8414x; 1.0547x over previous
import functools
import math

import jax
import jax.numpy as jnp
from jax import lax
from jax.experimental import pallas as pl
from jax.experimental.pallas import tpu as pltpu
from jax.experimental.pallas import tpu_sc as plsc

VOCAB = 1000000
D_MODEL = 64
BATCH = 4096
HIST = 200

NC = 2
NS = 16
NW = NC * NS
B = BATCH * HIST
C = 128
G4 = 4
R = C * G4
RPW = B // NW
NCH = RPW // C
NG = RPW // R
NBUF = 3
SCALE = math.sqrt(float(D_MODEL))

VBLK = 16384
NBLK = (VOCAB + VBLK - 1) // VBLK
PK_ROWS = NBLK * (VBLK // 2)

_mesh = plsc.VectorSubcoreMesh(core_axis_name="c", subcore_axis_name="s")


def _untile_body(in_ref, out_ref):
    blk = in_ref[...]
    h = VBLK // 2
    m = jnp.concatenate([blk[:, :h], blk[:, h:]], axis=0)
    out_ref[...] = m.T * SCALE


_untile = pl.pallas_call(
    _untile_body,
    grid=(NBLK,),
    in_specs=[pl.BlockSpec((D_MODEL, VBLK), lambda i: (0, i))],
    out_specs=pl.BlockSpec((VBLK // 2, 2 * D_MODEL), lambda i: (i, 0)),
    out_shape=jax.ShapeDtypeStruct((PK_ROWS, 2 * D_MODEL), jnp.float32),
)


@functools.partial(
    pl.kernel,
    out_type=jax.ShapeDtypeStruct((B, 2 * D_MODEL), jnp.float32),
    mesh=_mesh,
    compiler_params=pltpu.CompilerParams(use_tc_tiling_on_sc=False),
    scratch_types=[
        pltpu.VMEM((NCH, C), jnp.int32),
        pltpu.VMEM((NBUF, R, D_MODEL), jnp.float32),
        pltpu.SemaphoreType.DMA,
        pltpu.SemaphoreType.DMA,
        pltpu.SemaphoreType.DMA,
        pltpu.SemaphoreType.DMA,
        pltpu.SemaphoreType.DMA,
        pltpu.SemaphoreType.DMA,
    ],
)
def _emb_kernel(lut_hbm, idx_hbm, out_hbm, idx_v, rows_v, g0, g1, g2, s0, s1, s2):
    gsem = (g0, g1, g2)
    ssem = (s0, s1, s2)
    wid = lax.axis_index("s") * NC + lax.axis_index("c")
    row0 = wid * RPW

    pltpu.sync_copy(idx_hbm.at[pl.ds(wid * NCH, NCH)], idx_v)

    def fire_group(g, b):
        for j in range(G4):
            pltpu.async_copy(
                lut_hbm.at[idx_v.at[g * G4 + j]],
                rows_v.at[b, pl.ds(j * C, C)],
                gsem[b],
            )

    def wait_group(g, b):
        for j in range(G4):
            pltpu.make_async_copy(
                lut_hbm.at[idx_v.at[g * G4 + j]],
                rows_v.at[b, pl.ds(j * C, C)],
                gsem[b],
            ).wait()

    def fire_scatter(g, b):
        pltpu.async_copy(
            rows_v.at[b],
            out_hbm.at[pl.ds(row0 + g * R, R), pl.ds(0, D_MODEL)],
            ssem[b],
        )

    def wait_scatter(g, b):
        pltpu.make_async_copy(
            rows_v.at[b],
            out_hbm.at[pl.ds(row0 + g * R, R), pl.ds(0, D_MODEL)],
            ssem[b],
        ).wait()

    fire_group(0, 0)
    fire_group(1, 1)

    def step(t, _):
        for b in range(NBUF):
            g = t * NBUF + b

            @pl.when(g < NG)
            def _():
                wait_group(g, b)
                fire_scatter(g, b)
                bn = (b + 2) % NBUF

                @pl.when(jnp.logical_and(g >= 1, g + 2 < NG))
                def _():
                    wait_scatter(g - 1, bn)

                @pl.when(g + 2 < NG)
                def _():
                    fire_group(g + 2, bn)
        return 0

    lax.fori_loop(0, (NG + NBUF - 1) // NBUF, step, 0)

    for g in (NG - 3, NG - 2, NG - 1):
        wait_scatter(g, g % NBUF)


def kernel(x, lut):
    lut_pk = _untile(jnp.transpose(lut))
    lut2 = lut_pk.reshape(2 * PK_ROWS, D_MODEL)
    xi = x.astype(jnp.int32)
    r = ((xi >> 14) << 14) | ((xi & 8191) << 1) | ((xi >> 13) & 1)
    idx2d = r.reshape(B // C, C)
    out = _emb_kernel(lut2, idx2d)
    return out[:, :D_MODEL].reshape(BATCH, HIST, D_MODEL)


# speedup vs baseline: 2.8735x; 1.0113x over previous
import functools
import math

import jax
import jax.numpy as jnp
from jax import lax
from jax.experimental import pallas as pl
from jax.experimental.pallas import tpu as pltpu
from jax.experimental.pallas import tpu_sc as plsc

VOCAB = 1000000
D_MODEL = 64
BATCH = 4096
HIST = 200

NC = 2
NS = 16
NW = NC * NS
B = BATCH * HIST
C = 128
G4 = 4
R = C * G4
RPW = B // NW
NCH = RPW // C
NG = RPW // R
NBUF = 3
SCALE = math.sqrt(float(D_MODEL))

VBLK = 32768
NBLK = (VOCAB + VBLK - 1) // VBLK
PK_ROWS = NBLK * (VBLK // 2)

_mesh = plsc.VectorSubcoreMesh(core_axis_name="c", subcore_axis_name="s")


def _untile_body(in_ref, out_ref):
    blk = in_ref[...]
    h = VBLK // 2
    m = jnp.concatenate([blk[:, :h], blk[:, h:]], axis=0)
    out_ref[...] = m.T * SCALE


_untile = pl.pallas_call(
    _untile_body,
    grid=(NBLK,),
    in_specs=[pl.BlockSpec((D_MODEL, VBLK), lambda i: (0, i))],
    out_specs=pl.BlockSpec((VBLK // 2, 2 * D_MODEL), lambda i: (i, 0)),
    out_shape=jax.ShapeDtypeStruct((PK_ROWS, 2 * D_MODEL), jnp.float32),
)


@functools.partial(
    pl.kernel,
    out_type=jax.ShapeDtypeStruct((B, 2 * D_MODEL), jnp.float32),
    mesh=_mesh,
    compiler_params=pltpu.CompilerParams(use_tc_tiling_on_sc=False),
    scratch_types=[
        pltpu.VMEM((NCH, C), jnp.int32),
        pltpu.VMEM((NBUF, R, D_MODEL), jnp.float32),
        pltpu.SemaphoreType.DMA,
        pltpu.SemaphoreType.DMA,
        pltpu.SemaphoreType.DMA,
        pltpu.SemaphoreType.DMA,
        pltpu.SemaphoreType.DMA,
        pltpu.SemaphoreType.DMA,
    ],
)
def _emb_kernel(lut_hbm, idx_hbm, out_hbm, idx_v, rows_v, g0, g1, g2, s0, s1, s2):
    gsem = (g0, g1, g2)
    ssem = (s0, s1, s2)
    wid = lax.axis_index("s") * NC + lax.axis_index("c")
    row0 = wid * RPW

    pltpu.sync_copy(idx_hbm.at[pl.ds(wid * NCH, NCH)], idx_v)

    def fire_group(g, b):
        for j in range(G4):
            pltpu.async_copy(
                lut_hbm.at[idx_v.at[g * G4 + j]],
                rows_v.at[b, pl.ds(j * C, C)],
                gsem[b],
            )

    def wait_group(g, b):
        for j in range(G4):
            pltpu.make_async_copy(
                lut_hbm.at[idx_v.at[g * G4 + j]],
                rows_v.at[b, pl.ds(j * C, C)],
                gsem[b],
            ).wait()

    def fire_scatter(g, b):
        pltpu.async_copy(
            rows_v.at[b],
            out_hbm.at[pl.ds(row0 + g * R, R), pl.ds(0, D_MODEL)],
            ssem[b],
        )

    def wait_scatter(g, b):
        pltpu.make_async_copy(
            rows_v.at[b],
            out_hbm.at[pl.ds(row0 + g * R, R), pl.ds(0, D_MODEL)],
            ssem[b],
        ).wait()

    fire_group(0, 0)
    fire_group(1, 1)

    def step(t, _):
        for b in range(NBUF):
            g = t * NBUF + b

            @pl.when(g < NG)
            def _():
                wait_group(g, b)
                fire_scatter(g, b)
                bn = (b + 2) % NBUF

                @pl.when(jnp.logical_and(g >= 1, g + 2 < NG))
                def _():
                    wait_scatter(g - 1, bn)

                @pl.when(g + 2 < NG)
                def _():
                    fire_group(g + 2, bn)
        return 0

    lax.fori_loop(0, (NG + NBUF - 1) // NBUF, step, 0)

    for g in (NG - 3, NG - 2, NG - 1):
        wait_scatter(g, g % NBUF)


def kernel(x, lut):
    lut_pk = _untile(jnp.transpose(lut))
    lut2 = lut_pk.reshape(2 * PK_ROWS, D_MODEL)
    xi = x.astype(jnp.int32)
    r = ((xi >> 15) << 15) | ((xi & 16383) << 1) | ((xi >> 14) & 1)
    idx2d = r.reshape(B // C, C)
    out = _emb_kernel(lut2, idx2d)
    return out[:, :D_MODEL].reshape(BATCH, HIST, D_MODEL)
